# baseline (speedup 1.0000x reference)
import functools

import jax
import jax.numpy as jnp
from jax import lax
from jax.experimental import pallas as pl
from jax.experimental.pallas import tpu as pltpu
from jax.experimental.pallas import tpu_sc as plsc

EMB_DIM = 32
BATCH = 16384
NUM_NEG = 5

NUM_CORES = 2
NUM_SUBCORES = 16
NUM_WORKERS = NUM_CORES * NUM_SUBCORES
B_PER_W = BATCH // NUM_WORKERS
CHUNK = 128
POS_CHUNKS = B_PER_W // CHUNK
NEG_CHUNKS = B_PER_W * NUM_NEG // CHUNK
GROUPS = B_PER_W // 16


def _sc_scores(pos_u2d, pos_v2d, neg2d, u_weight, v_weight):
  mesh = plsc.VectorSubcoreMesh(core_axis_name="c", subcore_axis_name="s")

  @functools.partial(
      pl.kernel,
      mesh=mesh,
      compiler_params=pltpu.CompilerParams(
          needs_layout_passes=False, use_tc_tiling_on_sc=False),
      out_type=[
          jax.ShapeDtypeStruct((BATCH,), jnp.float32),
          jax.ShapeDtypeStruct((NUM_NEG * BATCH,), jnp.float32),
      ],
      scratch_types=[
          pltpu.VMEM((B_PER_W,), jnp.int32),
          pltpu.VMEM((B_PER_W,), jnp.int32),
          pltpu.VMEM((B_PER_W * NUM_NEG,), jnp.int32),
          pltpu.VMEM((B_PER_W, EMB_DIM), jnp.float32),
          pltpu.VMEM((B_PER_W, EMB_DIM), jnp.float32),
          pltpu.VMEM((B_PER_W * NUM_NEG, EMB_DIM), jnp.float32),
          pltpu.VMEM((B_PER_W,), jnp.float32),
          pltpu.VMEM((NUM_NEG * B_PER_W,), jnp.float32),
          pltpu.SemaphoreType.DMA,
      ],
  )
  def k(pos_u_hbm, pos_v_hbm, neg_hbm, u_w_hbm, v_w_hbm,
        out_s_hbm, out_n_hbm,
        idx_u, idx_v, idx_n, u_rows, v_rows, n_rows, s_out, n_out, sem):
    wid = lax.axis_index("s") * NUM_CORES + lax.axis_index("c")

    pltpu.sync_copy(pos_u_hbm.at[pl.ds(wid * B_PER_W, B_PER_W)], idx_u)
    pltpu.sync_copy(pos_v_hbm.at[pl.ds(wid * B_PER_W, B_PER_W)], idx_v)
    pltpu.sync_copy(neg_hbm.at[pl.ds(wid * B_PER_W * NUM_NEG, B_PER_W * NUM_NEG)], idx_n)

    copies = []
    for c in range(POS_CHUNKS):
      copies.append(pltpu.async_copy(
          u_w_hbm.at[idx_u.at[pl.ds(c * CHUNK, CHUNK)]],
          u_rows.at[pl.ds(c * CHUNK, CHUNK)], sem))
      copies.append(pltpu.async_copy(
          v_w_hbm.at[idx_v.at[pl.ds(c * CHUNK, CHUNK)]],
          v_rows.at[pl.ds(c * CHUNK, CHUNK)], sem))
    for c in range(NEG_CHUNKS):
      copies.append(pltpu.async_copy(
          v_w_hbm.at[idx_n.at[pl.ds(c * CHUNK, CHUNK)]],
          n_rows.at[pl.ds(c * CHUNK, CHUNK)], sem))
    for cp in copies:
      cp.wait()

    def group_body(g, carry):
      rows = g * 16 + lax.iota(jnp.int32, 16)
      acc_p = jnp.zeros((16,), jnp.float32)
      accs = [jnp.zeros((16,), jnp.float32) for _ in range(NUM_NEG)]
      nrows = rows * NUM_NEG
      for d in range(EMB_DIM):
        dvec = jnp.full((16,), d, jnp.int32)
        uu = plsc.load_gather(u_rows, [rows, dvec])
        vv = plsc.load_gather(v_rows, [rows, dvec])
        acc_p = acc_p + uu * vv
        for n in range(NUM_NEG):
          nn = plsc.load_gather(n_rows, [nrows + n, dvec])
          accs[n] = accs[n] + uu * nn
      s_out[pl.ds(g * 16, 16)] = acc_p
      for n in range(NUM_NEG):
        n_out[pl.ds(n * B_PER_W + g * 16, 16)] = accs[n]
      return carry

    lax.fori_loop(0, GROUPS, group_body, 0)

    pltpu.sync_copy(s_out, out_s_hbm.at[pl.ds(wid * B_PER_W, B_PER_W)])
    for n in range(NUM_NEG):
      pltpu.sync_copy(
          n_out.at[pl.ds(n * B_PER_W, B_PER_W)],
          out_n_hbm.at[pl.ds(n * BATCH + wid * B_PER_W, B_PER_W)])

  return k(pos_u2d, pos_v2d, neg2d, u_weight, v_weight)


def _softplus(x):
  return jnp.maximum(x, 0.0) + jnp.log(1.0 + jnp.exp(-jnp.abs(x)))


def _tc_loss_body(s_ref, n_ref, o_ref):
  s = jnp.clip(s_ref[...], -10.0, 10.0)
  n = jnp.clip(n_ref[...], -10.0, 10.0)
  total = jnp.sum(_softplus(-s)) + jnp.sum(_softplus(n))
  o_ref[...] = jnp.broadcast_to(total / BATCH, (1, 1))


def _tc_loss(score2d, neg2d):
  out = pl.pallas_call(
      _tc_loss_body,
      out_shape=jax.ShapeDtypeStruct((1, 1), jnp.float32),
  )(score2d, neg2d)
  return out[0, 0]


def kernel(pos_u, pos_v, neg_v, u_weight, v_weight):
  pos_u1d = pos_u.astype(jnp.int32).reshape(BATCH)
  pos_v1d = pos_v.astype(jnp.int32).reshape(BATCH)
  neg1d = neg_v.astype(jnp.int32).reshape(BATCH * NUM_NEG)
  score, neg_score = _sc_scores(pos_u1d, pos_v1d, neg1d, u_weight, v_weight)
  return _tc_loss(score.reshape(CHUNK, BATCH // CHUNK),
                  neg_score.reshape(NUM_NEG * BATCH // CHUNK, CHUNK))
